# baseline (speedup 1.0000x reference)
import jax
import jax.numpy as jnp
from jax.experimental import pallas as pl


def _add_kernel(x_ref, tab_ref, o_ref):
    o_ref[...] = x_ref[...] + tab_ref[...]


def kernel(x, pos_emb_table):
    batch, seq, dim = x.shape
    blk_b = 2
    blk_s = 512
    grid = (batch // blk_b, seq // blk_s)
    return pl.pallas_call(
        _add_kernel,
        grid=grid,
        in_specs=[
            pl.BlockSpec((blk_b, blk_s, dim), lambda b, s: (b, s, 0)),
            pl.BlockSpec((blk_s, dim), lambda b, s: (s, 0)),
        ],
        out_specs=pl.BlockSpec((blk_b, blk_s, dim), lambda b, s: (b, s, 0)),
        out_shape=jax.ShapeDtypeStruct(x.shape, x.dtype),
    )(x, pos_emb_table)


# speedup vs baseline: 1.2097x; 1.2097x over previous
import jax
import jax.numpy as jnp
from jax.experimental import pallas as pl


def _add_kernel(x_ref, tab_ref, o_ref):
    o_ref[...] = x_ref[...] + tab_ref[...]


def kernel(x, pos_emb_table):
    batch, seq, dim = x.shape
    blk_b = 2
    grid = (batch // blk_b,)
    return pl.pallas_call(
        _add_kernel,
        grid=grid,
        in_specs=[
            pl.BlockSpec((blk_b, seq, dim), lambda b: (b, 0, 0)),
            pl.BlockSpec((seq, dim), lambda b: (0, 0)),
        ],
        out_specs=pl.BlockSpec((blk_b, seq, dim), lambda b: (b, 0, 0)),
        out_shape=jax.ShapeDtypeStruct(x.shape, x.dtype),
    )(x, pos_emb_table)
